# baseline (speedup 1.0000x reference)
import functools

import jax
import jax.numpy as jnp
from jax import lax
from jax.experimental import pallas as pl
from jax.experimental.pallas import tpu as pltpu
from jax.experimental.pallas import tpu_sc as plsc

N = 10000
E = 320000
H = 10
C = 10
D = 100
CH = 64
NCH = E // CH
NW = 32


def _leaky(t):
    return jnp.where(t > 0.0, t, 0.2 * t)


def _pack2(a, b):
    return jnp.bitwise_or(jnp.bitwise_and(a, 0xFFFF), lax.shift_left(b, 16))


def _prep_body(h_ref, Ms_ref, Md_ref, R_ref, E_ref,
               S_ref, T_ref, F_ref, sc_ref):
    h = h_ref[...]
    Ms, Md, R, Emap = Ms_ref[...], Md_ref[...], R_ref[...], E_ref[...]
    a_s = jnp.dot(h, Ms, preferred_element_type=jnp.float32)
    a_d = jnp.dot(h, Md, preferred_element_type=jnp.float32)
    A = jnp.max(a_s, axis=0, keepdims=True)
    m = _leaky(A + a_d)
    sa = 16000.0 / jnp.maximum(
        jnp.maximum(jnp.max(jnp.abs(a_s)), jnp.max(jnp.abs(a_d))),
        jnp.maximum(jnp.max(jnp.abs(m)), 1e-20))
    inv_sa = 1.0 / sa
    sh = 16000.0 / jnp.maximum(
        jnp.maximum(jnp.max(jnp.abs(h)), 1.0), 1e-20)
    inv_sh = 1.0 / sh
    asq = jnp.round(a_s * sa)
    adq = jnp.round(a_d * sa)
    mq = jnp.round(m * sa)
    ex_self = jnp.exp(_leaky(a_s + a_d) - mq * inv_sa)
    msg_self = h * jnp.dot(ex_self, R, preferred_element_type=jnp.float32)
    z6 = jnp.zeros((N, 6), jnp.float32)
    z12 = jnp.zeros((N, 12), jnp.float32)
    onesq = jnp.full((N, 10), 1.0, jnp.float32) * jnp.round(sh)
    payload_q = jnp.concatenate([onesq, z6, jnp.round(h * sh), z12], axis=1)
    S = _pack2(jnp.dot(asq, Emap,
                       preferred_element_type=jnp.float32).astype(jnp.int32),
               payload_q.astype(jnp.int32))
    T = _pack2(jnp.dot(adq, Emap,
                       preferred_element_type=jnp.float32).astype(jnp.int32),
               jnp.dot(mq, Emap,
                       preferred_element_type=jnp.float32).astype(jnp.int32))
    S_ref[...] = S
    T_ref[...] = T
    F_ref[...] = jnp.concatenate([ex_self, z6, msg_self, z12], axis=1)
    sc_ref[...] = jnp.concatenate(
        [jnp.full((1, 16), inv_sa, jnp.float32),
         jnp.full((1, 16), 1.0, jnp.float32) * (1.0 / jnp.round(sh)),
         jnp.zeros((1, 96), jnp.float32)], axis=1)


def _combine(parts, slf, R, b):
    u = parts[0] + parts[1] + slf
    denom = jnp.dot(u[:, 0:10], R, preferred_element_type=jnp.float32) + 1e-16
    return jnp.maximum(u[:, 16:116] / denom + b, 0.0)


def _tc_h1_body(x_ref, W_ref, h_ref):
    h_ref[...] = jnp.dot(x_ref[...], W_ref[...],
                         preferred_element_type=jnp.float32)


def _tc_combine_body(parts_ref, Fp_ref, b_ref, W_ref, R_ref, h_ref):
    xn = _combine(parts_ref[...], Fp_ref[...], R_ref[...], b_ref[...])
    h_ref[...] = jnp.dot(xn, W_ref[...], preferred_element_type=jnp.float32)


def _tc_final_body(parts_ref, Fp_ref, b_ref, Wm_ref, bm_ref, R_ref, out_ref):
    xn = _combine(parts_ref[...], Fp_ref[...], R_ref[...], b_ref[...])
    g = jnp.mean(xn, axis=0, keepdims=True)
    out_ref[...] = jnp.dot(g, Wm_ref[...], preferred_element_type=jnp.float32) + bm_ref[...]


_tables_out = (
    jax.ShapeDtypeStruct((N, 128), jnp.int32),
    jax.ShapeDtypeStruct((N, 128), jnp.int32),
    jax.ShapeDtypeStruct((N, 128), jnp.float32),
    jax.ShapeDtypeStruct((1, 128), jnp.float32),
)

_tc_prep = pl.pallas_call(_prep_body, out_shape=_tables_out)
_tc_h1 = pl.pallas_call(
    _tc_h1_body, out_shape=jax.ShapeDtypeStruct((N, D), jnp.float32))
_tc_combine = pl.pallas_call(
    _tc_combine_body, out_shape=jax.ShapeDtypeStruct((N, D), jnp.float32))
_tc_final = pl.pallas_call(
    _tc_final_body, out_shape=jax.ShapeDtypeStruct((1, 10), jnp.float32))


def _sc_body(S_hbm, T_hbm, src_hbm, dst_hbm, scl_hbm, out_hbm,
             sidx, didx, rows, drows, msg, sclv, acc, sem_s, sem_t):
    c = lax.axis_index("c")
    s = lax.axis_index("s")
    w = s * 2 + c

    def _zb(i, carry):
        for k in range(8):
            msg[i, 16 * k:16 * (k + 1)] = jnp.zeros((16,), jnp.float32)
        return carry
    lax.fori_loop(0, CH, _zb, 0)
    for jj in range(10):
        cid = s + 16 * jj

        @pl.when(cid <= 155)
        def _():
            pltpu.sync_copy(msg, acc.at[pl.ds(cid * 64, 64)])

    @pl.when(s == 15)
    def _():
        pltpu.sync_copy(msg.at[pl.ds(0, 16)], acc.at[pl.ds(9984, 16)])
    pltpu.sync_copy(scl_hbm, sclv)
    inv_sa = sclv[0, 0:16]
    inv_sh = sclv[0, 16:32]
    plsc.subcore_barrier()

    nch = jnp.where(w < NCH % NW, NCH // NW + 1, NCH // NW)
    c0 = w * (NCH // NW) + jnp.minimum(w, NCH % NW)

    def _step(ci, carry):
        b = ci % 2
        b1 = 1 - b

        @pl.when(ci < nch)
        def _():
            eb = (c0 + ci) * CH
            pltpu.sync_copy(src_hbm.at[pl.ds(eb, CH)], sidx.at[b])
            pltpu.sync_copy(dst_hbm.at[pl.ds(eb, CH)], didx.at[b])
            pltpu.async_copy(S_hbm.at[sidx.at[b]], rows.at[b], sem_s.at[b])
            pltpu.async_copy(T_hbm.at[didx.at[b]], drows.at[b], sem_t.at[b])

        @pl.when(ci > 0)
        def _():
            pltpu.make_async_copy(S_hbm.at[sidx.at[b1]], rows.at[b1],
                                  sem_s.at[b1]).wait()
            pltpu.make_async_copy(T_hbm.at[didx.at[b1]], drows.at[b1],
                                  sem_t.at[b1]).wait()

            @plsc.parallel_loop(0, CH, step=1, unroll=4)
            def _edge(i):
                for k in range(8):
                    su = rows[b1, i, 16 * k:16 * (k + 1)]
                    tu = drows[b1, i, 16 * k:16 * (k + 1)]
                    asq = lax.shift_right_arithmetic(lax.shift_left(su, 16), 16)
                    hq = lax.shift_right_arithmetic(su, 16)
                    adq = lax.shift_right_arithmetic(lax.shift_left(tu, 16), 16)
                    mq = lax.shift_right_arithmetic(tu, 16)
                    t_f = (asq + adq).astype(jnp.float32) * inv_sa
                    m_f = mq.astype(jnp.float32) * inv_sa
                    ex = jnp.exp(_leaky(t_f) - m_f)
                    hk = hq.astype(jnp.float32) * inv_sh
                    msg[i, 16 * k:16 * (k + 1)] = hk * ex
            pltpu.sync_copy(msg, acc.at[didx.at[b1]], add=True)
        return carry
    lax.fori_loop(0, nch + 1, _step, 0)

    plsc.subcore_barrier()
    for jj in range(10):
        cid = s + 16 * jj

        @pl.when(cid <= 155)
        def _():
            pltpu.sync_copy(acc.at[pl.ds(cid * 64, 64)],
                            out_hbm.at[c, pl.ds(cid * 64, 64)])

    @pl.when(s == 15)
    def _():
        pltpu.sync_copy(acc.at[pl.ds(9984, 16)],
                        out_hbm.at[c, pl.ds(9984, 16)])


_sc_edge = functools.partial(
    pl.kernel,
    mesh=plsc.VectorSubcoreMesh(core_axis_name="c", subcore_axis_name="s"),
    out_type=jax.ShapeDtypeStruct((2, N, 128), jnp.float32),
    scratch_types=[
        pltpu.VMEM((2, CH), jnp.int32),
        pltpu.VMEM((2, CH), jnp.int32),
        pltpu.VMEM((2, CH, 128), jnp.int32),
        pltpu.VMEM((2, CH, 128), jnp.int32),
        pltpu.VMEM((CH, 128), jnp.float32),
        pltpu.VMEM((1, 128), jnp.float32),
        pltpu.VMEM_SHARED((N, 128), jnp.float32),
        pltpu.SemaphoreType.DMA((2,)),
        pltpu.SemaphoreType.DMA((2,)),
    ],
)(_sc_body)


def kernel(x, edge_index, W1, att_s1, att_d1, b1, W2, att_s2, att_d2, b2,
           W3, att_s3, att_d3, b3, Wm, bm):
    src = edge_index[0]
    dst = edge_index[1]
    cols = jnp.arange(D, dtype=jnp.int32)
    heads = cols // C
    R = jnp.zeros((H, D), jnp.float32).at[heads, cols].set(1.0)

    def mk_M(att):
        return jnp.zeros((D, H), jnp.float32).at[cols, heads].set(att.reshape(D))

    g = jnp.arange(128)
    hmap = jnp.where(g < 10, g, jnp.clip((g - 16) // 10, 0, 9))
    valid = (g < 10) | ((g >= 16) & (g < 116))
    Emap = jnp.where(valid[None, :] & (hmap[None, :] == jnp.arange(H)[:, None]),
                     1.0, 0.0).astype(jnp.float32)

    h = _tc_h1(x, W1)
    S, T, F, scl = _tc_prep(h, mk_M(att_s1), mk_M(att_d1), R, Emap)
    parts = _sc_edge(S, T, src, dst, scl)
    h = _tc_combine(parts, F, b1.reshape(1, D), W2, R)
    S, T, F, scl = _tc_prep(h, mk_M(att_s2), mk_M(att_d2), R, Emap)
    parts = _sc_edge(S, T, src, dst, scl)
    h = _tc_combine(parts, F, b2.reshape(1, D), W3, R)
    S, T, F, scl = _tc_prep(h, mk_M(att_s3), mk_M(att_d3), R, Emap)
    parts = _sc_edge(S, T, src, dst, scl)
    return _tc_final(parts, F, b3.reshape(1, D), Wm, bm.reshape(1, 10), R)


# speedup vs baseline: 1.4088x; 1.4088x over previous
import functools

import jax
import jax.numpy as jnp
from jax import lax
from jax.experimental import pallas as pl
from jax.experimental.pallas import tpu as pltpu
from jax.experimental.pallas import tpu_sc as plsc

N = 10000
E = 320000
H = 10
C = 10
D = 100
CH = 64
NCH = E // CH
NW = 32


def _leaky(t):
    return jnp.where(t > 0.0, t, 0.2 * t)


def _prep(h, Ms, Md, R, Emap):
    a_s = jnp.dot(h, Ms, preferred_element_type=jnp.float32)
    a_d = jnp.dot(h, Md, preferred_element_type=jnp.float32)
    A = (jnp.max(a_s, axis=0, keepdims=True) +
         jnp.max(a_d, axis=0, keepdims=True))
    ex_self = jnp.exp(_leaky(a_s + a_d) - A)
    z6 = jnp.zeros((N, 6), jnp.float32)
    z12 = jnp.zeros((N, 12), jnp.float32)
    msg_self = h * jnp.dot(ex_self, R, preferred_element_type=jnp.float32)
    asw = jnp.dot(a_s, Emap, preferred_element_type=jnp.float32)
    ones = jnp.ones((N, 10), jnp.float32)
    S = jnp.concatenate([asw, ones, z6, h, z12], axis=1)
    T = jnp.dot(a_d, Emap, preferred_element_type=jnp.float32)
    F = jnp.concatenate([ex_self, z6, msg_self, z12], axis=1)
    Aexp = jnp.dot(A, Emap, preferred_element_type=jnp.float32)
    return S, T, F, Aexp


def _combine(parts, slf, R, b):
    u = parts[0] + parts[1] + slf
    denom = jnp.dot(u[:, 0:10], R, preferred_element_type=jnp.float32) + 1e-16
    return jnp.maximum(u[:, 16:116] / denom + b, 0.0)


def _tc_first_body(x_ref, W_ref, Ms_ref, Md_ref, R_ref, E_ref,
                   S_ref, T_ref, F_ref, A_ref):
    h = jnp.dot(x_ref[...], W_ref[...], preferred_element_type=jnp.float32)
    S, T, F, A = _prep(h, Ms_ref[...], Md_ref[...], R_ref[...], E_ref[...])
    S_ref[...] = S
    T_ref[...] = T
    F_ref[...] = F
    A_ref[...] = A


def _tc_mid_body(parts_ref, Fp_ref, b_ref, W_ref, Ms_ref, Md_ref, R_ref,
                 E_ref, S_ref, T_ref, F_ref, A_ref):
    xn = _combine(parts_ref[...], Fp_ref[...], R_ref[...], b_ref[...])
    h = jnp.dot(xn, W_ref[...], preferred_element_type=jnp.float32)
    S, T, F, A = _prep(h, Ms_ref[...], Md_ref[...], R_ref[...], E_ref[...])
    S_ref[...] = S
    T_ref[...] = T
    F_ref[...] = F
    A_ref[...] = A


def _tc_final_body(parts_ref, Fp_ref, b_ref, Wm_ref, bm_ref, R_ref, out_ref):
    xn = _combine(parts_ref[...], Fp_ref[...], R_ref[...], b_ref[...])
    g = jnp.mean(xn, axis=0, keepdims=True)
    out_ref[...] = jnp.dot(g, Wm_ref[...], preferred_element_type=jnp.float32) + bm_ref[...]


_tables_out = (
    jax.ShapeDtypeStruct((N, 256), jnp.float32),
    jax.ShapeDtypeStruct((N, 128), jnp.float32),
    jax.ShapeDtypeStruct((N, 128), jnp.float32),
    jax.ShapeDtypeStruct((1, 128), jnp.float32),
)

_tc_first = pl.pallas_call(_tc_first_body, out_shape=_tables_out)
_tc_mid = pl.pallas_call(_tc_mid_body, out_shape=_tables_out)
_tc_final = pl.pallas_call(
    _tc_final_body, out_shape=jax.ShapeDtypeStruct((1, 10), jnp.float32))


def _sc_body(S_hbm, T_hbm, src_hbm, dst_hbm, av_hbm, out_hbm,
             sidx, didx, rows, drows, avm, acc, sem_s, sem_t):
    c = lax.axis_index("c")
    s = lax.axis_index("s")
    w = s * 2 + c

    def _zb(i, carry):
        for k in range(8):
            drows[0, i, 16 * k:16 * (k + 1)] = jnp.zeros((16,), jnp.float32)
        return carry
    lax.fori_loop(0, CH, _zb, 0)
    for jj in range(10):
        cid = s + 16 * jj

        @pl.when(cid <= 155)
        def _():
            pltpu.sync_copy(drows.at[0], acc.at[pl.ds(cid * 64, 64)])

    @pl.when(s == 15)
    def _():
        pltpu.sync_copy(drows.at[0, pl.ds(0, 16)], acc.at[pl.ds(9984, 16)])
    pltpu.sync_copy(av_hbm, avm)
    av = [avm[0, 16 * k:16 * (k + 1)] for k in range(8)]
    plsc.subcore_barrier()

    nch = jnp.where(w < NCH % NW, NCH // NW + 1, NCH // NW)
    c0 = w * (NCH // NW) + jnp.minimum(w, NCH % NW)

    def _step(ci, carry):
        b = ci % 2
        b1 = 1 - b

        @pl.when(ci < nch)
        def _():
            eb = (c0 + ci) * CH
            pltpu.sync_copy(src_hbm.at[pl.ds(eb, CH)], sidx.at[b])
            pltpu.sync_copy(dst_hbm.at[pl.ds(eb, CH)], didx.at[b])
            pltpu.async_copy(S_hbm.at[sidx.at[b]], rows.at[b], sem_s.at[b])
            pltpu.async_copy(T_hbm.at[didx.at[b]], drows.at[b], sem_t.at[b])

        @pl.when(ci > 0)
        def _():
            pltpu.make_async_copy(S_hbm.at[sidx.at[b1]], rows.at[b1],
                                  sem_s.at[b1]).wait()
            pltpu.make_async_copy(T_hbm.at[didx.at[b1]], drows.at[b1],
                                  sem_t.at[b1]).wait()

            @plsc.parallel_loop(0, CH, step=1, unroll=4)
            def _edge(i):
                for k in range(8):
                    asw = rows[b1, i, 16 * k:16 * (k + 1)]
                    adw = drows[b1, i, 16 * k:16 * (k + 1)]
                    ex = jnp.exp(_leaky(asw + adw) - av[k])
                    hk = rows[b1, i, 128 + 16 * k:128 + 16 * (k + 1)]
                    drows[b1, i, 16 * k:16 * (k + 1)] = hk * ex
            pltpu.sync_copy(drows.at[b1], acc.at[didx.at[b1]], add=True)
        return carry
    lax.fori_loop(0, nch + 1, _step, 0)

    plsc.subcore_barrier()
    for jj in range(10):
        cid = s + 16 * jj

        @pl.when(cid <= 155)
        def _():
            pltpu.sync_copy(acc.at[pl.ds(cid * 64, 64)],
                            out_hbm.at[c, pl.ds(cid * 64, 64)])

    @pl.when(s == 15)
    def _():
        pltpu.sync_copy(acc.at[pl.ds(9984, 16)],
                        out_hbm.at[c, pl.ds(9984, 16)])


_sc_edge = functools.partial(
    pl.kernel,
    mesh=plsc.VectorSubcoreMesh(core_axis_name="c", subcore_axis_name="s"),
    out_type=jax.ShapeDtypeStruct((2, N, 128), jnp.float32),
    scratch_types=[
        pltpu.VMEM((2, CH), jnp.int32),
        pltpu.VMEM((2, CH), jnp.int32),
        pltpu.VMEM((2, CH, 256), jnp.float32),
        pltpu.VMEM((2, CH, 128), jnp.float32),
        pltpu.VMEM((1, 128), jnp.float32),
        pltpu.VMEM_SHARED((N, 128), jnp.float32),
        pltpu.SemaphoreType.DMA((2,)),
        pltpu.SemaphoreType.DMA((2,)),
    ],
)(_sc_body)


def kernel(x, edge_index, W1, att_s1, att_d1, b1, W2, att_s2, att_d2, b2,
           W3, att_s3, att_d3, b3, Wm, bm):
    src = edge_index[0]
    dst = edge_index[1]
    cols = jnp.arange(D, dtype=jnp.int32)
    heads = cols // C
    R = jnp.zeros((H, D), jnp.float32).at[heads, cols].set(1.0)

    def mk_M(att):
        return jnp.zeros((D, H), jnp.float32).at[cols, heads].set(att.reshape(D))

    g = jnp.arange(128)
    hmap = jnp.where(g < 10, g, jnp.clip((g - 16) // 10, 0, 9))
    valid = (g < 10) | ((g >= 16) & (g < 116))
    Emap = jnp.where(valid[None, :] & (hmap[None, :] == jnp.arange(H)[:, None]),
                     1.0, 0.0).astype(jnp.float32)

    S, T, F, A = _tc_first(x, W1, mk_M(att_s1), mk_M(att_d1), R, Emap)
    parts = _sc_edge(S, T, src, dst, A)
    S, T, F, A = _tc_mid(parts, F, b1.reshape(1, D), W2, mk_M(att_s2),
                         mk_M(att_d2), R, Emap)
    parts = _sc_edge(S, T, src, dst, A)
    S, T, F, A = _tc_mid(parts, F, b2.reshape(1, D), W3, mk_M(att_s3),
                         mk_M(att_d3), R, Emap)
    parts = _sc_edge(S, T, src, dst, A)
    return _tc_final(parts, F, b3.reshape(1, D), Wm, bm.reshape(1, 10), R)


# speedup vs baseline: 1.4170x; 1.0058x over previous
import functools

import jax
import jax.numpy as jnp
from jax import lax
from jax.experimental import pallas as pl
from jax.experimental.pallas import tpu as pltpu
from jax.experimental.pallas import tpu_sc as plsc

N = 10000
E = 320000
H = 10
C = 10
D = 100
CH = 64
NCH = E // CH
NW = 32


def _leaky(t):
    return jnp.maximum(t, 0.2 * t)


def _prep(h, Ms, Md, R, Emap):
    a_s = jnp.dot(h, Ms, preferred_element_type=jnp.float32)
    a_d = jnp.dot(h, Md, preferred_element_type=jnp.float32)
    A = (jnp.max(a_s, axis=0, keepdims=True) +
         jnp.max(a_d, axis=0, keepdims=True))
    ex_self = jnp.exp(_leaky(a_s + a_d) - A)
    z6 = jnp.zeros((N, 6), jnp.float32)
    z12 = jnp.zeros((N, 12), jnp.float32)
    msg_self = h * jnp.dot(ex_self, R, preferred_element_type=jnp.float32)
    asw = jnp.dot(a_s, Emap, preferred_element_type=jnp.float32)
    ones = jnp.ones((N, 10), jnp.float32)
    S = jnp.concatenate([asw, ones, z6, h, z12], axis=1)
    T = jnp.dot(a_d, Emap, preferred_element_type=jnp.float32)
    F = jnp.concatenate([ex_self, z6, msg_self, z12], axis=1)
    Aexp = jnp.dot(A, Emap, preferred_element_type=jnp.float32)
    return S, T, F, Aexp


def _combine(parts, slf, R, b):
    u = parts[0] + parts[1] + slf
    denom = jnp.dot(u[:, 0:10], R, preferred_element_type=jnp.float32) + 1e-16
    return jnp.maximum(u[:, 16:116] / denom + b, 0.0)


def _tc_first_body(x_ref, W_ref, Ms_ref, Md_ref, R_ref, E_ref,
                   S_ref, T_ref, F_ref, A_ref):
    h = jnp.dot(x_ref[...], W_ref[...], preferred_element_type=jnp.float32)
    S, T, F, A = _prep(h, Ms_ref[...], Md_ref[...], R_ref[...], E_ref[...])
    S_ref[...] = S
    T_ref[...] = T
    F_ref[...] = F
    A_ref[...] = A


def _tc_mid_body(parts_ref, Fp_ref, b_ref, W_ref, Ms_ref, Md_ref, R_ref,
                 E_ref, S_ref, T_ref, F_ref, A_ref):
    xn = _combine(parts_ref[...], Fp_ref[...], R_ref[...], b_ref[...])
    h = jnp.dot(xn, W_ref[...], preferred_element_type=jnp.float32)
    S, T, F, A = _prep(h, Ms_ref[...], Md_ref[...], R_ref[...], E_ref[...])
    S_ref[...] = S
    T_ref[...] = T
    F_ref[...] = F
    A_ref[...] = A


def _tc_final_body(parts_ref, Fp_ref, b_ref, Wm_ref, bm_ref, R_ref, out_ref):
    xn = _combine(parts_ref[...], Fp_ref[...], R_ref[...], b_ref[...])
    g = jnp.mean(xn, axis=0, keepdims=True)
    out_ref[...] = jnp.dot(g, Wm_ref[...], preferred_element_type=jnp.float32) + bm_ref[...]


_tables_out = (
    jax.ShapeDtypeStruct((N, 256), jnp.float32),
    jax.ShapeDtypeStruct((N, 128), jnp.float32),
    jax.ShapeDtypeStruct((N, 128), jnp.float32),
    jax.ShapeDtypeStruct((1, 128), jnp.float32),
)

_tc_first = pl.pallas_call(_tc_first_body, out_shape=_tables_out)
_tc_mid = pl.pallas_call(_tc_mid_body, out_shape=_tables_out)
_tc_final = pl.pallas_call(
    _tc_final_body, out_shape=jax.ShapeDtypeStruct((1, 10), jnp.float32))


def _sc_body(S_hbm, T_hbm, src_hbm, dst_hbm, av_hbm, out_hbm,
             sidx, didx, rows, drows, avm, acc, sem_s, sem_t):
    c = lax.axis_index("c")
    s = lax.axis_index("s")
    w = s * 2 + c

    def _zb(i, carry):
        for k in range(8):
            drows[0, i, 16 * k:16 * (k + 1)] = jnp.zeros((16,), jnp.float32)
        return carry
    lax.fori_loop(0, CH, _zb, 0)
    for jj in range(10):
        cid = s + 16 * jj

        @pl.when(cid <= 155)
        def _():
            pltpu.sync_copy(drows.at[0], acc.at[pl.ds(cid * 64, 64)])

    @pl.when(s == 15)
    def _():
        pltpu.sync_copy(drows.at[0, pl.ds(0, 16)], acc.at[pl.ds(9984, 16)])
    pltpu.sync_copy(av_hbm, avm)
    av = [avm[0, 16 * k:16 * (k + 1)] for k in range(8)]
    plsc.subcore_barrier()

    nch = jnp.where(w < NCH % NW, NCH // NW + 1, NCH // NW)
    c0 = w * (NCH // NW) + jnp.minimum(w, NCH % NW)

    def _step(ci, carry):
        b = ci % 2
        b1 = 1 - b

        @pl.when(ci < nch)
        def _():
            eb = (c0 + ci) * CH
            pltpu.sync_copy(src_hbm.at[pl.ds(eb, CH)], sidx.at[b])
            pltpu.sync_copy(dst_hbm.at[pl.ds(eb, CH)], didx.at[b])
            pltpu.async_copy(S_hbm.at[sidx.at[b]], rows.at[b], sem_s.at[b])
            pltpu.async_copy(T_hbm.at[didx.at[b]], drows.at[b], sem_t.at[b])

        @pl.when(ci > 0)
        def _():
            pltpu.make_async_copy(S_hbm.at[sidx.at[b1]], rows.at[b1],
                                  sem_s.at[b1]).wait()
            pltpu.make_async_copy(T_hbm.at[didx.at[b1]], drows.at[b1],
                                  sem_t.at[b1]).wait()

            @plsc.parallel_loop(0, CH, step=1, unroll=4)
            def _edge(i):
                for k in range(8):
                    asw = rows[b1, i, 16 * k:16 * (k + 1)]
                    adw = drows[b1, i, 16 * k:16 * (k + 1)]
                    ex = jnp.exp(_leaky(asw + adw) - av[k])
                    hk = rows[b1, i, 128 + 16 * k:128 + 16 * (k + 1)]
                    drows[b1, i, 16 * k:16 * (k + 1)] = hk * ex
            pltpu.sync_copy(drows.at[b1], acc.at[didx.at[b1]], add=True)
        return carry
    lax.fori_loop(0, nch + 1, _step, 0)

    plsc.subcore_barrier()
    for jj in range(10):
        cid = s + 16 * jj

        @pl.when(cid <= 155)
        def _():
            pltpu.sync_copy(acc.at[pl.ds(cid * 64, 64)],
                            out_hbm.at[c, pl.ds(cid * 64, 64)])

    @pl.when(s == 15)
    def _():
        pltpu.sync_copy(acc.at[pl.ds(9984, 16)],
                        out_hbm.at[c, pl.ds(9984, 16)])


_sc_edge = functools.partial(
    pl.kernel,
    mesh=plsc.VectorSubcoreMesh(core_axis_name="c", subcore_axis_name="s"),
    out_type=jax.ShapeDtypeStruct((2, N, 128), jnp.float32),
    scratch_types=[
        pltpu.VMEM((2, CH), jnp.int32),
        pltpu.VMEM((2, CH), jnp.int32),
        pltpu.VMEM((2, CH, 256), jnp.float32),
        pltpu.VMEM((2, CH, 128), jnp.float32),
        pltpu.VMEM((1, 128), jnp.float32),
        pltpu.VMEM_SHARED((N, 128), jnp.float32),
        pltpu.SemaphoreType.DMA((2,)),
        pltpu.SemaphoreType.DMA((2,)),
    ],
)(_sc_body)


def kernel(x, edge_index, W1, att_s1, att_d1, b1, W2, att_s2, att_d2, b2,
           W3, att_s3, att_d3, b3, Wm, bm):
    src = edge_index[0]
    dst = edge_index[1]
    cols = jnp.arange(D, dtype=jnp.int32)
    heads = cols // C
    R = jnp.zeros((H, D), jnp.float32).at[heads, cols].set(1.0)

    def mk_M(att):
        return jnp.zeros((D, H), jnp.float32).at[cols, heads].set(att.reshape(D))

    g = jnp.arange(128)
    hmap = jnp.where(g < 10, g, jnp.clip((g - 16) // 10, 0, 9))
    valid = (g < 10) | ((g >= 16) & (g < 116))
    Emap = jnp.where(valid[None, :] & (hmap[None, :] == jnp.arange(H)[:, None]),
                     1.0, 0.0).astype(jnp.float32)

    S, T, F, A = _tc_first(x, W1, mk_M(att_s1), mk_M(att_d1), R, Emap)
    parts = _sc_edge(S, T, src, dst, A)
    S, T, F, A = _tc_mid(parts, F, b1.reshape(1, D), W2, mk_M(att_s2),
                         mk_M(att_d2), R, Emap)
    parts = _sc_edge(S, T, src, dst, A)
    S, T, F, A = _tc_mid(parts, F, b2.reshape(1, D), W3, mk_M(att_s3),
                         mk_M(att_d3), R, Emap)
    parts = _sc_edge(S, T, src, dst, A)
    return _tc_final(parts, F, b3.reshape(1, D), Wm, bm.reshape(1, 10), R)


# speedup vs baseline: 1.4202x; 1.0023x over previous
import functools

import jax
import jax.numpy as jnp
from jax import lax
from jax.experimental import pallas as pl
from jax.experimental.pallas import tpu as pltpu
from jax.experimental.pallas import tpu_sc as plsc

N = 10000
E = 320000
H = 10
C = 10
D = 100
CH = 64
NCH = E // CH
NW = 32


def _leaky(t):
    return jnp.maximum(t, 0.2 * t)


def _prep(h, Ms, Md, R, Emap):
    a_s = jnp.dot(h, Ms, preferred_element_type=jnp.float32)
    a_d = jnp.dot(h, Md, preferred_element_type=jnp.float32)
    A = (jnp.max(a_s, axis=0, keepdims=True) +
         jnp.max(a_d, axis=0, keepdims=True))
    ex_self = jnp.exp(_leaky(a_s + a_d) - A)
    z6 = jnp.zeros((N, 6), jnp.float32)
    z12 = jnp.zeros((N, 12), jnp.float32)
    msg_self = h * jnp.dot(ex_self, R, preferred_element_type=jnp.float32)
    asw = jnp.dot(a_s, Emap, preferred_element_type=jnp.float32)
    ones = jnp.ones((N, 10), jnp.float32)
    S = jnp.concatenate([asw, ones, z6, h, z12], axis=1)
    T = jnp.dot(a_d, Emap, preferred_element_type=jnp.float32)
    F = jnp.concatenate([ex_self, z6, msg_self, z12], axis=1)
    Aexp = jnp.dot(A, Emap, preferred_element_type=jnp.float32)
    return S, T, F, Aexp


def _combine(parts, slf, R, b):
    u = parts[0] + parts[1] + slf
    denom = jnp.dot(u[:, 0:10], R, preferred_element_type=jnp.float32) + 1e-16
    return jnp.maximum(u[:, 16:116] / denom + b, 0.0)


def _tc_first_body(x_ref, W_ref, Ms_ref, Md_ref, R_ref, E_ref,
                   S_ref, T_ref, F_ref, A_ref):
    h = jnp.dot(x_ref[...], W_ref[...], preferred_element_type=jnp.float32)
    S, T, F, A = _prep(h, Ms_ref[...], Md_ref[...], R_ref[...], E_ref[...])
    S_ref[...] = S
    T_ref[...] = T
    F_ref[...] = F
    A_ref[...] = A


def _tc_mid_body(parts_ref, Fp_ref, b_ref, W_ref, Ms_ref, Md_ref, R_ref,
                 E_ref, S_ref, T_ref, F_ref, A_ref):
    xn = _combine(parts_ref[...], Fp_ref[...], R_ref[...], b_ref[...])
    h = jnp.dot(xn, W_ref[...], preferred_element_type=jnp.float32)
    S, T, F, A = _prep(h, Ms_ref[...], Md_ref[...], R_ref[...], E_ref[...])
    S_ref[...] = S
    T_ref[...] = T
    F_ref[...] = F
    A_ref[...] = A


def _tc_final_body(parts_ref, Fp_ref, b_ref, Wm_ref, bm_ref, R_ref, out_ref):
    xn = _combine(parts_ref[...], Fp_ref[...], R_ref[...], b_ref[...])
    g = jnp.mean(xn, axis=0, keepdims=True)
    out_ref[...] = jnp.dot(g, Wm_ref[...], preferred_element_type=jnp.float32) + bm_ref[...]


_tables_out = (
    jax.ShapeDtypeStruct((N, 256), jnp.float32),
    jax.ShapeDtypeStruct((N, 128), jnp.float32),
    jax.ShapeDtypeStruct((N, 128), jnp.float32),
    jax.ShapeDtypeStruct((1, 128), jnp.float32),
)

_tc_first = pl.pallas_call(_tc_first_body, out_shape=_tables_out)
_tc_mid = pl.pallas_call(_tc_mid_body, out_shape=_tables_out)
_tc_final = pl.pallas_call(
    _tc_final_body, out_shape=jax.ShapeDtypeStruct((1, 10), jnp.float32))


def _sc_body(S_hbm, T_hbm, src_hbm, dst_hbm, av_hbm, out_hbm,
             sidx, didx, rows, drows, avm, acc, sem_s, sem_t, sem_sc):
    c = lax.axis_index("c")
    s = lax.axis_index("s")
    w = s * 2 + c

    def _zb(i, carry):
        for k in range(8):
            drows[0, i, 16 * k:16 * (k + 1)] = jnp.zeros((16,), jnp.float32)
        return carry
    lax.fori_loop(0, CH, _zb, 0)
    for jj in range(10):
        cid = s + 16 * jj

        @pl.when(cid <= 155)
        def _():
            pltpu.sync_copy(drows.at[0], acc.at[pl.ds(cid * 64, 64)])

    @pl.when(s == 15)
    def _():
        pltpu.sync_copy(drows.at[0, pl.ds(0, 16)], acc.at[pl.ds(9984, 16)])
    pltpu.sync_copy(av_hbm, avm)
    av = [avm[0, 16 * k:16 * (k + 1)] for k in range(8)]
    plsc.subcore_barrier()

    nch = jnp.where(w < NCH % NW, NCH // NW + 1, NCH // NW)
    c0 = w * (NCH // NW) + jnp.minimum(w, NCH % NW)

    def _step(ci, carry):
        b = ci % 2
        b1 = 1 - b

        @pl.when(ci < nch)
        def _():
            @pl.when(ci >= 2)
            def _():
                pltpu.make_async_copy(drows.at[b], acc.at[didx.at[b]],
                                      sem_sc.at[b]).wait()
            eb = (c0 + ci) * CH
            pltpu.sync_copy(src_hbm.at[pl.ds(eb, CH)], sidx.at[b])
            pltpu.sync_copy(dst_hbm.at[pl.ds(eb, CH)], didx.at[b])
            pltpu.async_copy(S_hbm.at[sidx.at[b]], rows.at[b], sem_s.at[b])
            pltpu.async_copy(T_hbm.at[didx.at[b]], drows.at[b], sem_t.at[b])

        @pl.when(ci > 0)
        def _():
            pltpu.make_async_copy(S_hbm.at[sidx.at[b1]], rows.at[b1],
                                  sem_s.at[b1]).wait()
            pltpu.make_async_copy(T_hbm.at[didx.at[b1]], drows.at[b1],
                                  sem_t.at[b1]).wait()

            @plsc.parallel_loop(0, CH, step=1, unroll=4)
            def _edge(i):
                for k in range(8):
                    asw = rows[b1, i, 16 * k:16 * (k + 1)]
                    adw = drows[b1, i, 16 * k:16 * (k + 1)]
                    ex = jnp.exp(_leaky(asw + adw) - av[k])
                    hk = rows[b1, i, 128 + 16 * k:128 + 16 * (k + 1)]
                    drows[b1, i, 16 * k:16 * (k + 1)] = hk * ex
            pltpu.async_copy(drows.at[b1], acc.at[didx.at[b1]],
                             sem_sc.at[b1], add=True)
        return carry
    lax.fori_loop(0, nch + 1, _step, 0)
    bl = (nch + 1) % 2

    @pl.when(nch >= 2)
    def _():
        pltpu.make_async_copy(drows.at[bl], acc.at[didx.at[bl]],
                              sem_sc.at[bl]).wait()

    @pl.when(nch >= 1)
    def _():
        pltpu.make_async_copy(drows.at[1 - bl], acc.at[didx.at[1 - bl]],
                              sem_sc.at[1 - bl]).wait()

    plsc.subcore_barrier()
    for jj in range(10):
        cid = s + 16 * jj

        @pl.when(cid <= 155)
        def _():
            pltpu.sync_copy(acc.at[pl.ds(cid * 64, 64)],
                            out_hbm.at[c, pl.ds(cid * 64, 64)])

    @pl.when(s == 15)
    def _():
        pltpu.sync_copy(acc.at[pl.ds(9984, 16)],
                        out_hbm.at[c, pl.ds(9984, 16)])


_sc_edge = functools.partial(
    pl.kernel,
    mesh=plsc.VectorSubcoreMesh(core_axis_name="c", subcore_axis_name="s"),
    out_type=jax.ShapeDtypeStruct((2, N, 128), jnp.float32),
    scratch_types=[
        pltpu.VMEM((2, CH), jnp.int32),
        pltpu.VMEM((2, CH), jnp.int32),
        pltpu.VMEM((2, CH, 256), jnp.float32),
        pltpu.VMEM((2, CH, 128), jnp.float32),
        pltpu.VMEM((1, 128), jnp.float32),
        pltpu.VMEM_SHARED((N, 128), jnp.float32),
        pltpu.SemaphoreType.DMA((2,)),
        pltpu.SemaphoreType.DMA((2,)),
        pltpu.SemaphoreType.DMA((2,)),
    ],
)(_sc_body)


def kernel(x, edge_index, W1, att_s1, att_d1, b1, W2, att_s2, att_d2, b2,
           W3, att_s3, att_d3, b3, Wm, bm):
    src = edge_index[0]
    dst = edge_index[1]
    cols = jnp.arange(D, dtype=jnp.int32)
    heads = cols // C
    R = jnp.zeros((H, D), jnp.float32).at[heads, cols].set(1.0)

    def mk_M(att):
        return jnp.zeros((D, H), jnp.float32).at[cols, heads].set(att.reshape(D))

    g = jnp.arange(128)
    hmap = jnp.where(g < 10, g, jnp.clip((g - 16) // 10, 0, 9))
    valid = (g < 10) | ((g >= 16) & (g < 116))
    Emap = jnp.where(valid[None, :] & (hmap[None, :] == jnp.arange(H)[:, None]),
                     1.0, 0.0).astype(jnp.float32)

    S, T, F, A = _tc_first(x, W1, mk_M(att_s1), mk_M(att_d1), R, Emap)
    parts = _sc_edge(S, T, src, dst, A)
    S, T, F, A = _tc_mid(parts, F, b1.reshape(1, D), W2, mk_M(att_s2),
                         mk_M(att_d2), R, Emap)
    parts = _sc_edge(S, T, src, dst, A)
    S, T, F, A = _tc_mid(parts, F, b2.reshape(1, D), W3, mk_M(att_s3),
                         mk_M(att_d3), R, Emap)
    parts = _sc_edge(S, T, src, dst, A)
    return _tc_final(parts, F, b3.reshape(1, D), Wm, bm.reshape(1, 10), R)
